# baseline (speedup 1.0000x reference)
import functools

import jax
import jax.numpy as jnp
from jax import lax
from jax.experimental import pallas as pl
from jax.experimental.pallas import tpu as pltpu
from jax.experimental.pallas import tpu_sc as plsc

VOCAB = 1_000_000
D = 64
BATCH = 4096
SEQ = 200
SCALE = 8.0

NC, NS, L = 2, 16, 16
NW = NC * NS
BB = BATCH // NW
NTILE = VOCAB // BB
NTAIL = VOCAB - NTILE * BB
TBASE = NTILE // NW
TREM = NTILE - TBASE * NW
_MESH = plsc.VectorSubcoreMesh(
    core_axis_name="c", subcore_axis_name="s",
    num_cores=NC, num_subcores=NS)


@functools.partial(
    pl.kernel,
    out_type=(jax.ShapeDtypeStruct((VOCAB * D,), jnp.float32),
              jax.ShapeDtypeStruct((NW, SEQ // 4, 4 * BB), jnp.int32)),
    mesh=_MESH,
    scratch_types=[
        pltpu.VMEM((SEQ, BB), jnp.int32),
        pltpu.VMEM((SEQ // 4, 4 * BB), jnp.int32),
        pltpu.VMEM((8, 8, 2 * BB), jnp.float32),
        pltpu.VMEM((8, 8, 2 * BB), jnp.float32),
        pltpu.VMEM((2 * BB * D,), jnp.float32),
        pltpu.VMEM((2 * BB * D,), jnp.float32),
        pltpu.VMEM((NTAIL * D,), jnp.float32),
        pltpu.SemaphoreType.DMA,
        pltpu.SemaphoreType.DMA,
        pltpu.SemaphoreType.DMA,
        pltpu.SemaphoreType.DMA,
    ],
    compiler_params=pltpu.CompilerParams(needs_layout_passes=False),
)
def _stage_sc(tab3_hbm, tok2_hbm, tail_hbm, tlin_hbm, tokshuf_hbm,
              tokv, tokf, tb0, tb1, rb0, rb1, tailv, sg0, sg1, so0, so1):
    wid = lax.axis_index("s") * NC + lax.axis_index("c")
    col0 = pl.multiple_of(wid * BB, BB)
    lanes = lax.iota(jnp.int32, L)

    pltpu.sync_copy(tok2_hbm.at[:, pl.ds(col0, BB)], tokv)

    @pl.loop(0, SEQ // 4)
    def _grp(g):
        for o in range(4 * BB // L):
            tokf[g, pl.ds(o * L, L)] = tokv[
                4 * g + o // (BB // L), pl.ds((o % (BB // L)) * L, L)]

    pltpu.sync_copy(tokf, tokshuf_hbm.at[wid])

    @pl.when(wid == NW - 1)
    def _():
        pltpu.sync_copy(tail_hbm, tailv)
        pltpu.sync_copy(tailv, tlin_hbm.at[pl.ds(NTILE * BB * D, NTAIL * D)])

    GB = 2 * BB
    NGRP = NTILE // 2
    GBASE = NGRP // NW
    ngrp = GBASE + jnp.where(wid < NGRP - GBASE * NW, 1, 0)

    def _fire_in(n, tb, sg):
        g = wid + n * NW
        pltpu.async_copy(
            tab3_hbm.at[:, :, pl.ds(pl.multiple_of(g * GB, GB), GB)], tb, sg)

    def _tpose(tb, rb):
        for q in range(D // L):
            c = lanes + q * L
            iv = c // 8
            rv = c % 8

            @plsc.parallel_loop(0, GB, unroll=16)
            def _row(b, tb=tb, rb=rb, iv=iv, rv=rv, q=q):
                bv = jnp.zeros((L,), jnp.int32) + b
                v = plsc.load_gather(tb, [iv, rv, bv])
                rb[pl.ds(b * D + q * L, L)] = v

    def _fire_out(n, rb, so):
        g = wid + n * NW
        pltpu.async_copy(
            rb, tlin_hbm.at[pl.ds(pl.multiple_of(g * GB * D, 8), GB * D)], so)

    _fire_in(0, tb0, sg0)
    _fire_in(1, tb1, sg1)

    @pl.loop(0, (GBASE + 1 + 1) // 2)
    def _pair(m):
        for p, (tb, rb, sg, so) in enumerate(
                ((tb0, rb0, sg0, so0), (tb1, rb1, sg1, so1))):
            n = 2 * m + p

            @pl.when(n < ngrp)
            def _(n=n, tb=tb, rb=rb, sg=sg, so=so):
                pltpu.make_async_copy(
                    tab3_hbm.at[:, :, pl.ds(0, GB)], tb, sg).wait()

                @pl.when(n >= 2)
                def _(rb=rb, so=so):
                    pltpu.make_async_copy(
                        rb, tlin_hbm.at[pl.ds(0, GB * D)], so).wait()

                _tpose(tb, rb)
                _fire_out(n, rb, so)

                @pl.when(n + 2 < ngrp)
                def _(n=n, tb=tb, sg=sg):
                    _fire_in(n + 2, tb, sg)

    pltpu.make_async_copy(rb0, tlin_hbm.at[pl.ds(0, GB * D)], so0).wait()
    pltpu.make_async_copy(rb1, tlin_hbm.at[pl.ds(0, GB * D)], so1).wait()


@functools.partial(
    pl.kernel,
    out_type=jax.ShapeDtypeStruct((SEQ * 8, NW, 8 * BB), jnp.float32),
    mesh=_MESH,
    scratch_types=[
        pltpu.VMEM((SEQ // 4, 4 * BB), jnp.int32),
        pltpu.VMEM((4 * BB, D), jnp.float32),
        pltpu.VMEM((4 * BB, D), jnp.float32),
        pltpu.VMEM((8, 8 * BB), jnp.float32),
        pltpu.VMEM((8, 8 * BB), jnp.float32),
        pltpu.SemaphoreType.DMA,
        pltpu.SemaphoreType.DMA,
        pltpu.SemaphoreType.DMA,
        pltpu.SemaphoreType.DMA,
    ],
    compiler_params=pltpu.CompilerParams(
        use_tc_tiling_on_sc=False, needs_layout_passes=False),
)
def _embed_sc(tokshuf_hbm, tlin_hbm, out_hbm, idxv, rows0, rows1,
              st0, st1, sg0, sg1, so0, so1):
    wid = lax.axis_index("s") * NC + lax.axis_index("c")
    lanes = lax.iota(jnp.int32, L)
    NGB = SEQ // 4

    pltpu.sync_copy(tokshuf_hbm.at[wid], idxv)

    def _gather(g, rows, sem):
        pltpu.async_copy(tlin_hbm.at[idxv.at[g]], rows, sem)

    def _tpose(rows, base, st):
        for b0 in range(BB // L):
            bv = lanes + (base + b0 * L)

            @plsc.parallel_loop(0, D, unroll=16)
            def _feat(c, rows=rows, st=st, bv=bv, b0=b0):
                cv = jnp.zeros((L,), jnp.int32) + c
                v = plsc.load_gather(rows, [bv, cv])
                st[c // 8, pl.ds((c % 8) * BB + b0 * L, L)] = v * SCALE

    def _fire_out(s, st, so):
        pltpu.async_copy(
            st, out_hbm.at[pl.ds(pl.multiple_of(s * 8, 8), 8), wid], so)

    def _drain_out(st, so):
        pltpu.make_async_copy(st, out_hbm.at[pl.ds(0, 8), 0], so).wait()

    _gather(0, rows0, sg0)
    _gather(1, rows1, sg1)

    @pl.loop(0, NGB // 2)
    def _pos(gg):
        for p, (rows, sg) in enumerate(((rows0, sg0), (rows1, sg1))):
            g = 2 * gg + p
            pltpu.make_async_copy(tlin_hbm.at[idxv.at[0]], rows, sg).wait()
            for sl in range(4):
                st, so = (st0, so0) if sl % 2 == 0 else (st1, so1)
                s = g * 4 + sl

                @pl.when(s >= 2)
                def _(st=st, so=so):
                    _drain_out(st, so)

                _tpose(rows, sl * BB, st)
                _fire_out(s, st, so)

            @pl.when(g < NGB - 2)
            def _(g=g, rows=rows, sg=sg):
                _gather(g + 2, rows, sg)

    _drain_out(st0, so0)
    _drain_out(st1, so1)


def kernel(tokens, table):
    tok2 = tokens.astype(jnp.int32).T
    tab3 = table.T.reshape(8, 8, VOCAB)
    tail = table[NTILE * BB:].reshape(-1)
    tlin, tokshuf = _stage_sc(tab3, tok2, tail)
    res5 = _embed_sc(tokshuf, tlin.reshape(VOCAB, D)).reshape(
        SEQ, 8, NW, 8, BB)
    return res5.transpose(2, 4, 0, 1, 3).reshape(BATCH, SEQ, D)


# speedup vs baseline: 1.2206x; 1.2206x over previous
import functools

import jax
import jax.numpy as jnp
from jax import lax
from jax.experimental import pallas as pl
from jax.experimental.pallas import tpu as pltpu
from jax.experimental.pallas import tpu_sc as plsc

VOCAB = 1_000_000
D = 64
BATCH = 4096
SEQ = 200
SCALE = 8.0

NC, NS, L = 2, 16, 16
NW = NC * NS
BB = BATCH // NW
NTILE = VOCAB // BB
NTAIL = VOCAB - NTILE * BB
TBASE = NTILE // NW
TREM = NTILE - TBASE * NW
_MESH = plsc.VectorSubcoreMesh(
    core_axis_name="c", subcore_axis_name="s",
    num_cores=NC, num_subcores=NS)


@functools.partial(
    pl.kernel,
    out_type=(jax.ShapeDtypeStruct((VOCAB * D,), jnp.float32),
              jax.ShapeDtypeStruct((NW, SEQ // 4, 4 * BB), jnp.int32)),
    mesh=_MESH,
    scratch_types=[
        pltpu.VMEM((SEQ, BB), jnp.int32),
        pltpu.VMEM((SEQ // 4, 4 * BB), jnp.int32),
        pltpu.VMEM((8, 8, 2 * BB), jnp.float32),
        pltpu.VMEM((8, 8, 2 * BB), jnp.float32),
        pltpu.VMEM((2 * BB * D,), jnp.float32),
        pltpu.VMEM((2 * BB * D,), jnp.float32),
        pltpu.VMEM((NTAIL * D,), jnp.float32),
        pltpu.SemaphoreType.DMA,
        pltpu.SemaphoreType.DMA,
        pltpu.SemaphoreType.DMA,
        pltpu.SemaphoreType.DMA,
    ],
    compiler_params=pltpu.CompilerParams(needs_layout_passes=False),
)
def _stage_sc(tab3_hbm, tok2_hbm, tail_hbm, tlin_hbm, tokshuf_hbm,
              tokv, tokf, tb0, tb1, rb0, rb1, tailv, sg0, sg1, so0, so1):
    wid = lax.axis_index("s") * NC + lax.axis_index("c")
    col0 = pl.multiple_of(wid * BB, BB)
    lanes = lax.iota(jnp.int32, L)

    pltpu.sync_copy(tok2_hbm.at[:, pl.ds(col0, BB)], tokv)

    @pl.loop(0, SEQ // 4)
    def _grp(g):
        for o in range(4 * BB // L):
            tokf[g, pl.ds(o * L, L)] = tokv[
                4 * g + o // (BB // L), pl.ds((o % (BB // L)) * L, L)]

    pltpu.sync_copy(tokf, tokshuf_hbm.at[wid])

    @pl.when(wid == NW - 1)
    def _():
        pltpu.sync_copy(tail_hbm, tailv)
        pltpu.sync_copy(tailv, tlin_hbm.at[pl.ds(NTILE * BB * D, NTAIL * D)])

    GB = 2 * BB
    NGRP = NTILE // 2
    GBASE = NGRP // NW
    ngrp = GBASE + jnp.where(wid < NGRP - GBASE * NW, 1, 0)

    def _fire_in(n, tb, sg):
        g = wid + n * NW
        pltpu.async_copy(
            tab3_hbm.at[:, :, pl.ds(pl.multiple_of(g * GB, GB), GB)], tb, sg)

    def _tpose(tb, rb):
        for q in range(D // L):
            c = lanes + q * L
            iv = c // 8
            rv = c % 8

            @plsc.parallel_loop(0, GB, unroll=16)
            def _row(b, tb=tb, rb=rb, iv=iv, rv=rv, q=q):
                bv = jnp.zeros((L,), jnp.int32) + b
                v = plsc.load_gather(tb, [iv, rv, bv])
                rb[pl.ds(b * D + q * L, L)] = v

    def _fire_out(n, rb, so):
        g = wid + n * NW
        pltpu.async_copy(
            rb, tlin_hbm.at[pl.ds(pl.multiple_of(g * GB * D, 8), GB * D)], so)

    _fire_in(0, tb0, sg0)
    _fire_in(1, tb1, sg1)

    @pl.loop(0, (GBASE + 1 + 1) // 2)
    def _pair(m):
        for p, (tb, rb, sg, so) in enumerate(
                ((tb0, rb0, sg0, so0), (tb1, rb1, sg1, so1))):
            n = 2 * m + p

            @pl.when(n < ngrp)
            def _(n=n, tb=tb, rb=rb, sg=sg, so=so):
                pltpu.make_async_copy(
                    tab3_hbm.at[:, :, pl.ds(0, GB)], tb, sg).wait()

                @pl.when(n >= 2)
                def _(rb=rb, so=so):
                    pltpu.make_async_copy(
                        rb, tlin_hbm.at[pl.ds(0, GB * D)], so).wait()

                _tpose(tb, rb)
                _fire_out(n, rb, so)

                @pl.when(n + 2 < ngrp)
                def _(n=n, tb=tb, sg=sg):
                    _fire_in(n + 2, tb, sg)

    pltpu.make_async_copy(rb0, tlin_hbm.at[pl.ds(0, GB * D)], so0).wait()
    pltpu.make_async_copy(rb1, tlin_hbm.at[pl.ds(0, GB * D)], so1).wait()


@functools.partial(
    pl.kernel,
    out_type=jax.ShapeDtypeStruct((SEQ * 8, NW, 8 * BB), jnp.float32),
    mesh=_MESH,
    scratch_types=[
        pltpu.VMEM((SEQ // 4, 4 * BB), jnp.int32),
        pltpu.VMEM((4 * BB, D), jnp.float32),
        pltpu.VMEM((4 * BB, D), jnp.float32),
        pltpu.VMEM((8, 8 * BB), jnp.float32),
        pltpu.VMEM((8, 8 * BB), jnp.float32),
        pltpu.SemaphoreType.DMA,
        pltpu.SemaphoreType.DMA,
        pltpu.SemaphoreType.DMA,
        pltpu.SemaphoreType.DMA,
    ],
    compiler_params=pltpu.CompilerParams(
        use_tc_tiling_on_sc=False, needs_layout_passes=False),
)
def _embed_sc(tokshuf_hbm, tlin_hbm, out_hbm, idxv, rows0, rows1,
              st0, st1, sg0, sg1, so0, so1):
    wid = lax.axis_index("s") * NC + lax.axis_index("c")
    lanes = lax.iota(jnp.int32, L)
    NGB = SEQ // 4

    pltpu.sync_copy(tokshuf_hbm.at[wid], idxv)

    def _gather(g, rows, sem):
        pltpu.async_copy(tlin_hbm.at[idxv.at[g]], rows, sem)

    def _tpose(rows, base, st):
        for b0 in range(BB // L):
            bv = lanes + (base + b0 * L)

            @plsc.parallel_loop(0, D, unroll=16)
            def _feat(c, rows=rows, st=st, bv=bv, b0=b0):
                cv = jnp.zeros((L,), jnp.int32) + c
                v = plsc.load_gather(rows, [bv, cv])
                st[c // 8, pl.ds((c % 8) * BB + b0 * L, L)] = v * SCALE

    def _fire_out(s, st, so):
        pltpu.async_copy(
            st, out_hbm.at[pl.ds(pl.multiple_of(s * 8, 8), 8), wid], so)

    def _drain_out(st, so):
        pltpu.make_async_copy(st, out_hbm.at[pl.ds(0, 8), 0], so).wait()

    _gather(0, rows0, sg0)
    _gather(1, rows1, sg1)

    @pl.loop(0, NGB // 2)
    def _pos(gg):
        for p, (rows, sg) in enumerate(((rows0, sg0), (rows1, sg1))):
            g = 2 * gg + p
            pltpu.make_async_copy(tlin_hbm.at[idxv.at[0]], rows, sg).wait()
            for sl in range(4):
                st, so = (st0, so0) if sl % 2 == 0 else (st1, so1)
                s = g * 4 + sl

                @pl.when(s >= 2)
                def _(st=st, so=so):
                    _drain_out(st, so)

                _tpose(rows, sl * BB, st)
                _fire_out(s, st, so)

            @pl.when(g < NGB - 2)
            def _(g=g, rows=rows, sg=sg):
                _gather(g + 2, rows, sg)

    _drain_out(st0, so0)
    _drain_out(st1, so1)


def kernel(tokens, table):
    tokshuf = (tokens.astype(jnp.int32).T
               .reshape(SEQ // 4, 4, NW, BB)
               .transpose(2, 0, 1, 3)
               .reshape(NW, SEQ // 4, 4 * BB))
    res5 = _embed_sc(tokshuf, table).reshape(SEQ, 8, NW, 8, BB)
    return res5.transpose(2, 4, 0, 1, 3).reshape(BATCH, SEQ, D)
